# baseline (speedup 1.0000x reference)
import jax
import jax.numpy as jnp
from jax import lax
from jax.experimental import pallas as pl
from jax.experimental.pallas import tpu as pltpu

FEAT = 512
CDIM = 256
K = 1024
B = 32
T = 128
NTOK = B * T
BLK = 1024
NBLK = NTOK // BLK
BPB = BLK // T


def _ln(h, g, b):
    m = jnp.mean(h, axis=-1, keepdims=True)
    v = jnp.mean((h - m) ** 2, axis=-1, keepdims=True)
    return (h - m) / jnp.sqrt(v + 1e-5) * g + b


def _body(x_ref, eW1, eb1, g1, b1, eW2, eb2, g2, b2, cb, cbt, cbb,
          dW1, db1, dW2, db2, pW1, pb1, pW2, pb2,
          idx_out, recon_out, commit_out, cbl_out, contr_out, total_out,
          pooled_sc, racc, cacc):
    i = pl.program_id(0)
    xb = x_ref[...]
    h = jnp.dot(xb, eW1[...], preferred_element_type=jnp.float32) + eb1[...]
    h = jnp.maximum(_ln(h, g1[...], b1[...]), 0.0)
    h = jnp.dot(h, eW2[...], preferred_element_type=jnp.float32) + eb2[...]
    enc = jnp.maximum(_ln(h, g2[...], b2[...]), 0.0)

    ct = cbt[...]
    s = jnp.dot(enc, ct, preferred_element_type=jnp.float32)
    cbsq = jnp.sum(ct * ct, axis=0, keepdims=True)
    d2 = cbsq - 2.0 * s
    mins = jnp.min(d2, axis=1, keepdims=True)
    mask = d2 <= mins
    col = lax.broadcasted_iota(jnp.int32, (BLK, K), 1)
    idx = jnp.min(jnp.where(mask, col, K), axis=1, keepdims=True)
    idx_out[...] = idx

    onehot = jnp.where(col == idx, 1.0, 0.0).astype(jnp.bfloat16)
    q = jnp.dot(onehot, cbb[...], preferred_element_type=jnp.float32)

    hd = jnp.maximum(
        jnp.dot(q.astype(jnp.bfloat16), dW1[...],
                preferred_element_type=jnp.float32) + db1[...], 0.0)
    r = jnp.dot(hd.astype(jnp.bfloat16), dW2[...],
                preferred_element_type=jnp.float32) + db2[...]
    rs = jnp.sum((r - xb) ** 2)
    cs = jnp.sum((enc - q) ** 2)

    @pl.when(i == 0)
    def _():
        racc[0, 0] = rs
        cacc[0, 0] = cs

    @pl.when(i > 0)
    def _():
        racc[0, 0] += rs
        cacc[0, 0] += cs

    rid = lax.broadcasted_iota(jnp.int32, (BPB, BLK), 0)
    cid = lax.broadcasted_iota(jnp.int32, (BPB, BLK), 1)
    M = jnp.where(cid // T == rid, 1.0 / T, 0.0).astype(jnp.bfloat16)
    counts = jnp.dot(M, onehot, preferred_element_type=jnp.float32)
    pooled_sc[pl.ds(i * BPB, BPB), :] = jnp.dot(
        counts, cb[...], preferred_element_type=jnp.float32)

    @pl.when(i == NBLK - 1)
    def _():
        pooled = pooled_sc[...]
        p = jnp.maximum(
            jnp.dot(pooled, pW1[...], preferred_element_type=jnp.float32)
            + pb1[...], 0.0)
        p = jnp.dot(p, pW2[...], preferred_element_type=jnp.float32) + pb2[...]
        nrm = jnp.maximum(jnp.sqrt(jnp.sum(p * p, axis=1, keepdims=True)), 1e-12)
        n = p / nrm
        sim = lax.dot_general(
            n, n, (((1,), (1,)), ((), ())),
            preferred_element_type=jnp.float32) / 0.1
        mx = jnp.max(sim, axis=1, keepdims=True)
        logp = sim - mx - jnp.log(jnp.sum(jnp.exp(sim - mx), axis=1,
                                          keepdims=True))
        er = lax.broadcasted_iota(jnp.int32, (B, B), 0)
        ec = lax.broadcasted_iota(jnp.int32, (B, B), 1)
        contr = -jnp.sum(jnp.where(er == ec, logp, 0.0)) / B
        recon = racc[0, 0] / (NTOK * FEAT)
        commit = cacc[0, 0] / (NTOK * CDIM)
        w = lambda ref, v: ref.__setitem__(
            (slice(None), slice(None)), jnp.broadcast_to(v, (1, 1)))
        w(recon_out, recon)
        w(commit_out, commit)
        w(cbl_out, commit)
        w(contr_out, contr)
        w(total_out, recon + commit * 0.25 + commit + contr * 0.1)


def kernel(x, enc_W1, enc_b1, ln1_g, ln1_b, enc_W2, enc_b2, ln2_g, ln2_b,
           codebook, dec_W1, dec_b1, dec_W2, dec_b2, proj_W1, proj_b1,
           proj_W2, proj_b2):
    xf = x.reshape(NTOK, FEAT)
    full = lambda a: pl.BlockSpec(a.shape, lambda i: (0,) * a.ndim)
    r2 = lambda a: a.reshape(1, -1)
    bf = lambda a: a.astype(jnp.bfloat16)
    args = (xf, enc_W1, r2(enc_b1), r2(ln1_g), r2(ln1_b), enc_W2, r2(enc_b2),
            r2(ln2_g), r2(ln2_b), codebook, codebook.T, bf(codebook),
            bf(dec_W1), r2(dec_b1), bf(dec_W2), r2(dec_b2),
            proj_W1, r2(proj_b1), proj_W2, r2(proj_b2))
    in_specs = [pl.BlockSpec((BLK, FEAT), lambda i: (i, 0))] + [
        full(a) for a in args[1:]]
    sc = pl.BlockSpec((1, 1), lambda i: (0, 0))
    out = pl.pallas_call(
        _body,
        grid=(NBLK,),
        in_specs=in_specs,
        out_specs=[pl.BlockSpec((BLK, 1), lambda i: (i, 0)),
                   sc, sc, sc, sc, sc],
        out_shape=[jax.ShapeDtypeStruct((NTOK, 1), jnp.int32)] + [
            jax.ShapeDtypeStruct((1, 1), jnp.float32)] * 5,
        scratch_shapes=[pltpu.VMEM((B, CDIM), jnp.float32),
                        pltpu.SMEM((1, 1), jnp.float32),
                        pltpu.SMEM((1, 1), jnp.float32)],
    )(*args)
    idx, recon, commit, cbl, contr, total = out
    return (idx.reshape(B, T), recon[0, 0], commit[0, 0], cbl[0, 0],
            contr[0, 0], total[0, 0])


# speedup vs baseline: 1.0173x; 1.0173x over previous
import jax
import jax.numpy as jnp
from jax import lax
from jax.experimental import pallas as pl
from jax.experimental.pallas import tpu as pltpu

FEAT = 512
CDIM = 256
K = 1024
B = 32
T = 128
NTOK = B * T
BLK = 1024
NBLK = NTOK // BLK
BPB = BLK // T


def _rowsum(a):
    ones = jnp.ones((a.shape[1], 1), jnp.float32)
    return jnp.dot(a, ones, preferred_element_type=jnp.float32)


def _ln(h, g, b):
    n = h.shape[1]
    m = _rowsum(h) * (1.0 / n)
    v = _rowsum((h - m) ** 2) * (1.0 / n)
    return (h - m) / jnp.sqrt(v + 1e-5) * g + b


def _body(x_ref, eW1, eb1, g1, b1, eW2, eb2, g2, b2, cb, cbt,
          dW1, db1, dW2, db2, pW1, pb1, pW2, pb2,
          idx_out, recon_out, commit_out, cbl_out, contr_out, total_out,
          pooled_sc, cbsq_sc, racc, cacc):
    i = pl.program_id(0)

    @pl.when(i == 0)
    def _():
        ct0 = cbt[...]
        cbsq_sc[...] = jnp.sum(ct0 * ct0, axis=0, keepdims=True)

    xb = x_ref[...]
    h = jnp.dot(xb, eW1[...], preferred_element_type=jnp.float32) + eb1[...]
    h = jnp.maximum(_ln(h, g1[...], b1[...]), 0.0)
    h = jnp.dot(h, eW2[...], preferred_element_type=jnp.float32) + eb2[...]
    enc = jnp.maximum(_ln(h, g2[...], b2[...]), 0.0)

    s2 = jnp.dot(enc * -2.0, cbt[...], preferred_element_type=jnp.float32)
    d2 = s2 + cbsq_sc[...]
    mins = jnp.min(d2, axis=1, keepdims=True)
    colf = lax.broadcasted_iota(jnp.int32, (BLK, K), 1).astype(jnp.float32)
    idxf = jnp.min(jnp.where(d2 <= mins, colf, jnp.float32(K)), axis=1,
                   keepdims=True)
    idx_out[...] = idxf.astype(jnp.int32)

    onehot = jnp.where(colf == idxf, 1.0, 0.0)
    q = jnp.dot(onehot, cb[...], preferred_element_type=jnp.float32)

    hd = jnp.maximum(
        jnp.dot(q, dW1[...], preferred_element_type=jnp.float32) + db1[...],
        0.0)
    r = jnp.dot(hd, dW2[...], preferred_element_type=jnp.float32) + db2[...]
    rs = jnp.sum(_rowsum((r - xb) ** 2))
    cs = jnp.sum(_rowsum((enc - q) ** 2))

    @pl.when(i == 0)
    def _():
        racc[0, 0] = rs
        cacc[0, 0] = cs

    @pl.when(i > 0)
    def _():
        racc[0, 0] += rs
        cacc[0, 0] += cs

    rid = lax.broadcasted_iota(jnp.int32, (BPB, BLK), 0)
    cid = lax.broadcasted_iota(jnp.int32, (BPB, BLK), 1)
    M = jnp.where(cid // T == rid, 1.0 / T, 0.0)
    pooled_sc[pl.ds(i * BPB, BPB), :] = jnp.dot(
        M, q, preferred_element_type=jnp.float32)

    @pl.when(i == NBLK - 1)
    def _():
        pooled = pooled_sc[...]
        p = jnp.maximum(
            jnp.dot(pooled, pW1[...], preferred_element_type=jnp.float32)
            + pb1[...], 0.0)
        p = jnp.dot(p, pW2[...], preferred_element_type=jnp.float32) + pb2[...]
        nrm = jnp.maximum(jnp.sqrt(jnp.sum(p * p, axis=1, keepdims=True)),
                          1e-12)
        n = p / nrm
        sim = lax.dot_general(
            n, n, (((1,), (1,)), ((), ())),
            preferred_element_type=jnp.float32) / 0.1
        mx = jnp.max(sim, axis=1, keepdims=True)
        logp = sim - mx - jnp.log(jnp.sum(jnp.exp(sim - mx), axis=1,
                                          keepdims=True))
        er = lax.broadcasted_iota(jnp.int32, (B, B), 0)
        ec = lax.broadcasted_iota(jnp.int32, (B, B), 1)
        contr = -jnp.sum(jnp.where(er == ec, logp, 0.0)) / B
        recon = racc[0, 0] / (NTOK * FEAT)
        commit = cacc[0, 0] / (NTOK * CDIM)
        w = lambda ref, v: ref.__setitem__(
            (slice(None), slice(None)), jnp.broadcast_to(v, (1, 1)))
        w(recon_out, recon)
        w(commit_out, commit)
        w(cbl_out, commit)
        w(contr_out, contr)
        w(total_out, recon + commit * 0.25 + commit + contr * 0.1)


def kernel(x, enc_W1, enc_b1, ln1_g, ln1_b, enc_W2, enc_b2, ln2_g, ln2_b,
           codebook, dec_W1, dec_b1, dec_W2, dec_b2, proj_W1, proj_b1,
           proj_W2, proj_b2):
    xf = x.reshape(NTOK, FEAT)
    full = lambda a: pl.BlockSpec(a.shape, lambda i: (0,) * a.ndim)
    r2 = lambda a: a.reshape(1, -1)
    args = (xf, enc_W1, r2(enc_b1), r2(ln1_g), r2(ln1_b), enc_W2, r2(enc_b2),
            r2(ln2_g), r2(ln2_b), codebook, codebook.T, dec_W1, r2(dec_b1),
            dec_W2, r2(dec_b2), proj_W1, r2(proj_b1), proj_W2, r2(proj_b2))
    in_specs = [pl.BlockSpec((BLK, FEAT), lambda i: (i, 0))] + [
        full(a) for a in args[1:]]
    sc = pl.BlockSpec((1, 1), lambda i: (0, 0))
    out = pl.pallas_call(
        _body,
        grid=(NBLK,),
        in_specs=in_specs,
        out_specs=[pl.BlockSpec((BLK, 1), lambda i: (i, 0)),
                   sc, sc, sc, sc, sc],
        out_shape=[jax.ShapeDtypeStruct((NTOK, 1), jnp.int32)] + [
            jax.ShapeDtypeStruct((1, 1), jnp.float32)] * 5,
        scratch_shapes=[pltpu.VMEM((B, CDIM), jnp.float32),
                        pltpu.VMEM((1, K), jnp.float32),
                        pltpu.SMEM((1, 1), jnp.float32),
                        pltpu.SMEM((1, 1), jnp.float32)],
    )(*args)
    idx, recon, commit, cbl, contr, total = out
    return (idx.reshape(B, T), recon[0, 0], commit[0, 0], cbl[0, 0],
            contr[0, 0], total[0, 0])


# speedup vs baseline: 1.0785x; 1.0602x over previous
import jax
import jax.numpy as jnp
from jax import lax
from jax.experimental import pallas as pl
from jax.experimental.pallas import tpu as pltpu

FEAT = 512
CDIM = 256
K = 1024
B = 32
T = 128
NTOK = B * T
BLK = 1024
NBLK = NTOK // BLK
BPB = BLK // T


def _ln(h, g, b):
    m = jnp.mean(h, axis=-1, keepdims=True)
    v = jnp.mean((h - m) ** 2, axis=-1, keepdims=True)
    return (h - m) / jnp.sqrt(v + 1e-5) * g + b


def _body(x_ref, eW1, eb1, g1, b1, eW2, eb2, g2, b2, cb, cbt,
          dW1, db1, dW2, db2, pW1, pb1, pW2, pb2,
          idx_out, recon_out, commit_out, cbl_out, contr_out, total_out,
          pooled_sc, cbsq_sc, racc, cacc):
    i = pl.program_id(0)

    @pl.when(i == 0)
    def _():
        ct0 = cbt[...]
        cbsq_sc[...] = jnp.sum(ct0 * ct0, axis=0, keepdims=True)

    xb = x_ref[...]
    h = jnp.dot(xb, eW1[...], preferred_element_type=jnp.float32) + eb1[...]
    h = jnp.maximum(_ln(h, g1[...], b1[...]), 0.0)
    h = jnp.dot(h, eW2[...], preferred_element_type=jnp.float32) + eb2[...]
    enc = jnp.maximum(_ln(h, g2[...], b2[...]), 0.0)

    s2 = jnp.dot(enc * -2.0, cbt[...], preferred_element_type=jnp.float32)
    d2 = s2 + cbsq_sc[...]
    mins = jnp.min(d2, axis=1, keepdims=True)
    col = lax.broadcasted_iota(jnp.int32, (BLK, K), 1)
    idx = jnp.min(jnp.where(d2 <= mins, col, K), axis=1, keepdims=True)
    idx_out[...] = idx

    onehot = jnp.where(col == idx, 1.0, 0.0)
    q = jnp.dot(onehot, cb[...], preferred_element_type=jnp.float32)

    hd = jnp.maximum(
        jnp.dot(q, dW1[...], preferred_element_type=jnp.float32) + db1[...],
        0.0)
    r = jnp.dot(hd, dW2[...], preferred_element_type=jnp.float32) + db2[...]
    rs = jnp.sum((r - xb) ** 2)
    cs = jnp.sum((enc - q) ** 2)

    @pl.when(i == 0)
    def _():
        racc[0, 0] = rs
        cacc[0, 0] = cs

    @pl.when(i > 0)
    def _():
        racc[0, 0] += rs
        cacc[0, 0] += cs

    rid = lax.broadcasted_iota(jnp.int32, (BPB, BLK), 0)
    cid = lax.broadcasted_iota(jnp.int32, (BPB, BLK), 1)
    M = jnp.where(cid // T == rid, 1.0 / T, 0.0)
    pooled_sc[pl.ds(i * BPB, BPB), :] = jnp.dot(
        M, q, preferred_element_type=jnp.float32)

    @pl.when(i == NBLK - 1)
    def _():
        pooled = pooled_sc[...]
        p = jnp.maximum(
            jnp.dot(pooled, pW1[...], preferred_element_type=jnp.float32)
            + pb1[...], 0.0)
        p = jnp.dot(p, pW2[...], preferred_element_type=jnp.float32) + pb2[...]
        nrm = jnp.maximum(jnp.sqrt(jnp.sum(p * p, axis=1, keepdims=True)),
                          1e-12)
        n = p / nrm
        sim = lax.dot_general(
            n, n, (((1,), (1,)), ((), ())),
            preferred_element_type=jnp.float32) / 0.1
        mx = jnp.max(sim, axis=1, keepdims=True)
        logp = sim - mx - jnp.log(jnp.sum(jnp.exp(sim - mx), axis=1,
                                          keepdims=True))
        er = lax.broadcasted_iota(jnp.int32, (B, B), 0)
        ec = lax.broadcasted_iota(jnp.int32, (B, B), 1)
        contr = -jnp.sum(jnp.where(er == ec, logp, 0.0)) / B
        recon = racc[0, 0] / (NTOK * FEAT)
        commit = cacc[0, 0] / (NTOK * CDIM)
        w = lambda ref, v: ref.__setitem__(
            (slice(None), slice(None)), jnp.broadcast_to(v, (1, 1)))
        w(recon_out, recon)
        w(commit_out, commit)
        w(cbl_out, commit)
        w(contr_out, contr)
        w(total_out, recon + commit * 0.25 + commit + contr * 0.1)


def kernel(x, enc_W1, enc_b1, ln1_g, ln1_b, enc_W2, enc_b2, ln2_g, ln2_b,
           codebook, dec_W1, dec_b1, dec_W2, dec_b2, proj_W1, proj_b1,
           proj_W2, proj_b2):
    xf = x.reshape(NTOK, FEAT)
    full = lambda a: pl.BlockSpec(a.shape, lambda i: (0,) * a.ndim)
    r2 = lambda a: a.reshape(1, -1)
    args = (xf, enc_W1, r2(enc_b1), r2(ln1_g), r2(ln1_b), enc_W2, r2(enc_b2),
            r2(ln2_g), r2(ln2_b), codebook, codebook.T, dec_W1, r2(dec_b1),
            dec_W2, r2(dec_b2), proj_W1, r2(proj_b1), proj_W2, r2(proj_b2))
    in_specs = [pl.BlockSpec((BLK, FEAT), lambda i: (i, 0))] + [
        full(a) for a in args[1:]]
    sc = pl.BlockSpec((1, 1), lambda i: (0, 0))
    out = pl.pallas_call(
        _body,
        grid=(NBLK,),
        in_specs=in_specs,
        out_specs=[pl.BlockSpec((BLK, 1), lambda i: (i, 0)),
                   sc, sc, sc, sc, sc],
        out_shape=[jax.ShapeDtypeStruct((NTOK, 1), jnp.int32)] + [
            jax.ShapeDtypeStruct((1, 1), jnp.float32)] * 5,
        scratch_shapes=[pltpu.VMEM((B, CDIM), jnp.float32),
                        pltpu.VMEM((1, K), jnp.float32),
                        pltpu.SMEM((1, 1), jnp.float32),
                        pltpu.SMEM((1, 1), jnp.float32)],
    )(*args)
    idx, recon, commit, cbl, contr, total = out
    return (idx.reshape(B, T), recon[0, 0], commit[0, 0], cbl[0, 0],
            contr[0, 0], total[0, 0])


# speedup vs baseline: 1.0795x; 1.0009x over previous
import jax
import jax.numpy as jnp
from jax import lax
from jax.experimental import pallas as pl
from jax.experimental.pallas import tpu as pltpu

FEAT = 512
CDIM = 256
K = 1024
B = 32
T = 128
NTOK = B * T
CHUNK = 1024
NCH = 2
BLK = CHUNK * NCH
NBLK = NTOK // BLK
CPB = CHUNK // T


def _ln(h, g, b):
    m = jnp.mean(h, axis=-1, keepdims=True)
    v = jnp.mean((h - m) ** 2, axis=-1, keepdims=True)
    return (h - m) / jnp.sqrt(v + 1e-5) * g + b


def _body(x_ref, eW1, eb1, g1, b1, eW2, eb2, g2, b2, cb, cbt,
          dW1, db1, dW2, db2, pW1, pb1, pW2, pb2,
          idx_out, recon_out, commit_out, cbl_out, contr_out, total_out,
          pooled_sc, cbsq_sc, racc, cacc):
    i = pl.program_id(0)

    @pl.when(i == 0)
    def _():
        ct0 = cbt[...]
        cbsq_sc[...] = jnp.sum(ct0 * ct0, axis=0, keepdims=True)

    cbsq = cbsq_sc[...]
    rs = 0.0
    cs = 0.0
    for c in range(NCH):
        xb = x_ref[c * CHUNK:(c + 1) * CHUNK, :]
        h = jnp.dot(xb, eW1[...], preferred_element_type=jnp.float32) + eb1[...]
        h = jnp.maximum(_ln(h, g1[...], b1[...]), 0.0)
        h = jnp.dot(h, eW2[...], preferred_element_type=jnp.float32) + eb2[...]
        enc = jnp.maximum(_ln(h, g2[...], b2[...]), 0.0)

        s2 = jnp.dot(enc * -2.0, cbt[...], preferred_element_type=jnp.float32)
        d2 = s2 + cbsq
        mins = jnp.min(d2, axis=1, keepdims=True)
        col = lax.broadcasted_iota(jnp.int32, (CHUNK, K), 1)
        idx = jnp.min(jnp.where(d2 <= mins, col, K), axis=1, keepdims=True)
        idx_out[c * CHUNK:(c + 1) * CHUNK, :] = idx

        onehot = jnp.where(col == idx, 1.0, 0.0)
        q = jnp.dot(onehot, cb[...], preferred_element_type=jnp.float32)

        hd = jnp.maximum(
            jnp.dot(q, dW1[...], preferred_element_type=jnp.float32)
            + db1[...], 0.0)
        r = jnp.dot(hd, dW2[...], preferred_element_type=jnp.float32) + db2[...]
        rs += jnp.sum((r - xb) ** 2)
        cs += jnp.sum((enc - q) ** 2)

        rid = lax.broadcasted_iota(jnp.int32, (CPB, CHUNK), 0)
        cid = lax.broadcasted_iota(jnp.int32, (CPB, CHUNK), 1)
        M = jnp.where(cid // T == rid, 1.0 / T, 0.0)
        pooled_sc[pl.ds(i * (NCH * CPB) + c * CPB, CPB), :] = jnp.dot(
            M, q, preferred_element_type=jnp.float32)

    @pl.when(i == 0)
    def _():
        racc[0, 0] = rs
        cacc[0, 0] = cs

    @pl.when(i > 0)
    def _():
        racc[0, 0] += rs
        cacc[0, 0] += cs

    @pl.when(i == NBLK - 1)
    def _():
        pooled = pooled_sc[...]
        p = jnp.maximum(
            jnp.dot(pooled, pW1[...], preferred_element_type=jnp.float32)
            + pb1[...], 0.0)
        p = jnp.dot(p, pW2[...], preferred_element_type=jnp.float32) + pb2[...]
        nrm = jnp.maximum(jnp.sqrt(jnp.sum(p * p, axis=1, keepdims=True)),
                          1e-12)
        n = p / nrm
        sim = lax.dot_general(
            n, n, (((1,), (1,)), ((), ())),
            preferred_element_type=jnp.float32) / 0.1
        mx = jnp.max(sim, axis=1, keepdims=True)
        logp = sim - mx - jnp.log(jnp.sum(jnp.exp(sim - mx), axis=1,
                                          keepdims=True))
        er = lax.broadcasted_iota(jnp.int32, (B, B), 0)
        ec = lax.broadcasted_iota(jnp.int32, (B, B), 1)
        contr = -jnp.sum(jnp.where(er == ec, logp, 0.0)) / B
        recon = racc[0, 0] / (NTOK * FEAT)
        commit = cacc[0, 0] / (NTOK * CDIM)
        w = lambda ref, v: ref.__setitem__(
            (slice(None), slice(None)), jnp.broadcast_to(v, (1, 1)))
        w(recon_out, recon)
        w(commit_out, commit)
        w(cbl_out, commit)
        w(contr_out, contr)
        w(total_out, recon + commit * 0.25 + commit + contr * 0.1)


def kernel(x, enc_W1, enc_b1, ln1_g, ln1_b, enc_W2, enc_b2, ln2_g, ln2_b,
           codebook, dec_W1, dec_b1, dec_W2, dec_b2, proj_W1, proj_b1,
           proj_W2, proj_b2):
    xf = x.reshape(NTOK, FEAT)
    full = lambda a: pl.BlockSpec(a.shape, lambda i: (0,) * a.ndim)
    r2 = lambda a: a.reshape(1, -1)
    args = (xf, enc_W1, r2(enc_b1), r2(ln1_g), r2(ln1_b), enc_W2, r2(enc_b2),
            r2(ln2_g), r2(ln2_b), codebook, codebook.T, dec_W1, r2(dec_b1),
            dec_W2, r2(dec_b2), proj_W1, r2(proj_b1), proj_W2, r2(proj_b2))
    in_specs = [pl.BlockSpec((BLK, FEAT), lambda i: (i, 0))] + [
        full(a) for a in args[1:]]
    sc = pl.BlockSpec((1, 1), lambda i: (0, 0))
    out = pl.pallas_call(
        _body,
        grid=(NBLK,),
        in_specs=in_specs,
        out_specs=[pl.BlockSpec((BLK, 1), lambda i: (i, 0)),
                   sc, sc, sc, sc, sc],
        out_shape=[jax.ShapeDtypeStruct((NTOK, 1), jnp.int32)] + [
            jax.ShapeDtypeStruct((1, 1), jnp.float32)] * 5,
        scratch_shapes=[pltpu.VMEM((B, CDIM), jnp.float32),
                        pltpu.VMEM((1, K), jnp.float32),
                        pltpu.SMEM((1, 1), jnp.float32),
                        pltpu.SMEM((1, 1), jnp.float32)],
    )(*args)
    idx, recon, commit, cbl, contr, total = out
    return (idx.reshape(B, T), recon[0, 0], commit[0, 0], cbl[0, 0],
            contr[0, 0], total[0, 0])


# speedup vs baseline: 1.3007x; 1.2049x over previous
import jax
import jax.numpy as jnp
from jax import lax
from jax.experimental import pallas as pl
from jax.experimental.pallas import tpu as pltpu

FEAT = 512
CDIM = 256
K = 1024
B = 32
T = 128
NTOK = B * T
CHUNK = 1024
NCH = 2
BLK = CHUNK * NCH
NBLK = NTOK // BLK
CPB = CHUNK // T

_DN = (((1,), (1,)), ((), ()))


def _ln(h, g, b):
    m = jnp.mean(h, axis=-1, keepdims=True)
    v = jnp.mean((h - m) ** 2, axis=-1, keepdims=True)
    return (h - m) / jnp.sqrt(v + 1e-5) * g + b


def _row(ref):
    return ref[...].reshape(1, -1)


def _body(x_ref, eW1, eb1, g1, b1, eW2, eb2, g2, b2, cb,
          dW1, db1, dW2, db2, pW1, pb1, pW2, pb2,
          idx_out, recon_out, commit_out, cbl_out, contr_out, total_out,
          pooled_sc, cbsq_sc, racc, cacc):
    i = pl.program_id(0)
    cbv = cb[...]

    @pl.when(i == 0)
    def _():
        ones = jnp.ones((1, CDIM), jnp.float32)
        cbsq_sc[...] = lax.dot_general(ones, cbv * cbv, _DN,
                                       preferred_element_type=jnp.float32)

    cbsq = cbsq_sc[...]
    rs = 0.0
    cs = 0.0
    for c in range(NCH):
        xb = x_ref[c * CHUNK:(c + 1) * CHUNK, :]
        h = jnp.dot(xb, eW1[...], preferred_element_type=jnp.float32) + _row(eb1)
        h = jnp.maximum(_ln(h, _row(g1), _row(b1)), 0.0)
        h = jnp.dot(h, eW2[...], preferred_element_type=jnp.float32) + _row(eb2)
        enc = jnp.maximum(_ln(h, _row(g2), _row(b2)), 0.0)

        s2 = lax.dot_general(enc * -2.0, cbv, _DN,
                             preferred_element_type=jnp.float32)
        d2 = s2 + cbsq
        mins = jnp.min(d2, axis=1, keepdims=True)
        col = lax.broadcasted_iota(jnp.int32, (CHUNK, K), 1)
        idx = jnp.min(jnp.where(d2 <= mins, col, K), axis=1, keepdims=True)
        idx_out[c * CHUNK:(c + 1) * CHUNK, :] = idx

        onehot = jnp.where(col == idx, 1.0, 0.0)
        q = jnp.dot(onehot, cbv, preferred_element_type=jnp.float32)

        hd = jnp.maximum(
            jnp.dot(q, dW1[...], preferred_element_type=jnp.float32)
            + _row(db1), 0.0)
        r = jnp.dot(hd, dW2[...], preferred_element_type=jnp.float32) + _row(db2)
        rs += jnp.sum((r - xb) ** 2)
        cs += jnp.sum((enc - q) ** 2)

        rid = lax.broadcasted_iota(jnp.int32, (CPB, CHUNK), 0)
        cid = lax.broadcasted_iota(jnp.int32, (CPB, CHUNK), 1)
        M = jnp.where(cid // T == rid, 1.0 / T, 0.0)
        pooled_sc[pl.ds(i * (NCH * CPB) + c * CPB, CPB), :] = jnp.dot(
            M, q, preferred_element_type=jnp.float32)

    @pl.when(i == 0)
    def _():
        racc[0, 0] = rs
        cacc[0, 0] = cs

    @pl.when(i > 0)
    def _():
        racc[0, 0] += rs
        cacc[0, 0] += cs

    @pl.when(i == NBLK - 1)
    def _():
        pooled = pooled_sc[...]
        p = jnp.maximum(
            jnp.dot(pooled, pW1[...], preferred_element_type=jnp.float32)
            + _row(pb1), 0.0)
        p = jnp.dot(p, pW2[...], preferred_element_type=jnp.float32) + _row(pb2)
        nrm = jnp.maximum(jnp.sqrt(jnp.sum(p * p, axis=1, keepdims=True)),
                          1e-12)
        n = p / nrm
        sim = lax.dot_general(n, n, _DN,
                              preferred_element_type=jnp.float32) / 0.1
        mx = jnp.max(sim, axis=1, keepdims=True)
        logp = sim - mx - jnp.log(jnp.sum(jnp.exp(sim - mx), axis=1,
                                          keepdims=True))
        er = lax.broadcasted_iota(jnp.int32, (B, B), 0)
        ec = lax.broadcasted_iota(jnp.int32, (B, B), 1)
        contr = -jnp.sum(jnp.where(er == ec, logp, 0.0)) / B
        recon = racc[0, 0] / (NTOK * FEAT)
        commit = cacc[0, 0] / (NTOK * CDIM)
        w = lambda ref, v: ref.__setitem__(
            (slice(None), slice(None)), jnp.broadcast_to(v, (1, 1)))
        w(recon_out, recon)
        w(commit_out, commit)
        w(cbl_out, commit)
        w(contr_out, contr)
        w(total_out, recon + commit * 0.25 + commit + contr * 0.1)


def kernel(x, enc_W1, enc_b1, ln1_g, ln1_b, enc_W2, enc_b2, ln2_g, ln2_b,
           codebook, dec_W1, dec_b1, dec_W2, dec_b2, proj_W1, proj_b1,
           proj_W2, proj_b2):
    xf = x.reshape(NTOK, FEAT)
    full = lambda a: pl.BlockSpec(a.shape, lambda i: (0,) * a.ndim)
    args = (xf, enc_W1, enc_b1, ln1_g, ln1_b, enc_W2, enc_b2,
            ln2_g, ln2_b, codebook, dec_W1, dec_b1,
            dec_W2, dec_b2, proj_W1, proj_b1, proj_W2, proj_b2)
    in_specs = [pl.BlockSpec((BLK, FEAT), lambda i: (i, 0))] + [
        full(a) for a in args[1:]]
    sc = pl.BlockSpec((1, 1), lambda i: (0, 0))
    out = pl.pallas_call(
        _body,
        grid=(NBLK,),
        in_specs=in_specs,
        out_specs=[pl.BlockSpec((BLK, 1), lambda i: (i, 0)),
                   sc, sc, sc, sc, sc],
        out_shape=[jax.ShapeDtypeStruct((NTOK, 1), jnp.int32)] + [
            jax.ShapeDtypeStruct((1, 1), jnp.float32)] * 5,
        scratch_shapes=[pltpu.VMEM((B, CDIM), jnp.float32),
                        pltpu.VMEM((1, K), jnp.float32),
                        pltpu.SMEM((1, 1), jnp.float32),
                        pltpu.SMEM((1, 1), jnp.float32)],
    )(*args)
    idx, recon, commit, cbl, contr, total = out
    return (idx.reshape(B, T), recon[0, 0], commit[0, 0], cbl[0, 0],
            contr[0, 0], total[0, 0])
